# Initial kernel scaffold; baseline (speedup 1.0000x reference)
import jax
import jax.numpy as jnp
from jax.experimental import pallas as pl


def kernel(heat, wh, reg, conf_thrs):
    raise NotImplementedError("write your pallas kernel here")


# speedup vs baseline: 9.1063x; 9.1063x over previous
import functools

import jax
import jax.numpy as jnp
from jax.experimental import pallas as pl
from jax.experimental.pallas import tpu as pltpu

_K = 100
_BIG = 2 ** 30


def _decode_kernel(heat_ref, wh_ref, reg_ref, conf_ref, out_ref,
                   s_ref, rmax_ref, score_s, cls_s, ys_s, xs_s):
    C, H, W = 80, 128, 128
    R = C * H

    sig = jax.nn.sigmoid(heat_ref[0].reshape(R, W))

    lane = jax.lax.broadcasted_iota(jnp.int32, (R, W), 1)
    row = jax.lax.broadcasted_iota(jnp.int32, (R, W), 0)

    zcol = jnp.zeros((R, 1), jnp.float32)
    wl = jnp.concatenate([sig[:, 1:], zcol], axis=1)
    wr = jnp.concatenate([zcol, sig[:, :-1]], axis=1)
    mw = jnp.maximum(sig, jnp.maximum(wl, wr))

    zrow = jnp.zeros((1, W), jnp.float32)
    hd = jnp.concatenate([mw[1:, :], zrow], axis=0)
    hd = jnp.where(row % H == H - 1, 0.0, hd)
    hu = jnp.concatenate([zrow, mw[:-1, :]], axis=0)
    hu = jnp.where(row % H == 0, 0.0, hu)
    hmax = jnp.maximum(mw, jnp.maximum(hd, hu))

    sup = jnp.where(hmax == sig, sig, 0.0)
    s_ref[...] = sup
    rmax_ref[...] = jnp.max(sup.reshape(C, H, W), axis=2)

    riota = jax.lax.broadcasted_iota(jnp.int32, (C, H), 0) * H + \
        jax.lax.broadcasted_iota(jnp.int32, (C, H), 1)
    lane1 = jax.lax.broadcasted_iota(jnp.int32, (1, W), 1)

    def select_body(k, _):
        rm = rmax_ref[...]
        m = jnp.max(rm)
        r = jnp.min(jnp.where(rm == m, riota, _BIG))
        vals = s_ref[pl.ds(r, 1), :]
        x = jnp.min(jnp.where(vals == m, lane1, _BIG))
        newrow = jnp.where(lane1 == x, 0.0, vals)
        s_ref[pl.ds(r, 1), :] = newrow
        rmax_ref[...] = jnp.where(riota == r, jnp.max(newrow), rm)
        score_s[k] = m
        cls_s[k] = r // H
        ys_s[k] = r % H
        xs_s[k] = x
        return 0

    jax.lax.fori_loop(0, _K, select_body, 0)

    def gather_body(k, carry):
        cls_v, sc_v, x1_v, y1_v, x2_v, y2_v = carry
        y = ys_s[k]
        x = xs_s[k]
        c = cls_s[k]
        sc = score_s[k]
        sel = (lane1 == x)
        w0 = jnp.sum(jnp.where(sel, wh_ref[0, 0, pl.ds(y, 1), :], 0.0))
        w1 = jnp.sum(jnp.where(sel, wh_ref[0, 1, pl.ds(y, 1), :], 0.0))
        r0 = jnp.sum(jnp.where(sel, reg_ref[0, 0, pl.ds(y, 1), :], 0.0))
        r1 = jnp.sum(jnp.where(sel, reg_ref[0, 1, pl.ds(y, 1), :], 0.0))
        xf = x.astype(jnp.float32) + r0
        yf = y.astype(jnp.float32) + r1
        hw = w0 * 0.5
        hh = w1 * 0.5
        thr = conf_ref[c]
        clsf = jnp.where(sc < thr, -1.0, c.astype(jnp.float32))
        put = lambda acc, v: jnp.where(lane1 == k, v, acc)
        return (put(cls_v, clsf), put(sc_v, sc),
                put(x1_v, xf - hw), put(y1_v, yf - hh),
                put(x2_v, xf + hw), put(y2_v, yf + hh))

    z = jnp.zeros((1, W), jnp.float32)
    outs = jax.lax.fori_loop(0, _K, gather_body, (z, z, z, z, z, z))
    out_ref[0] = jnp.concatenate(list(outs) + [z, z], axis=0)


@jax.jit
def kernel(heat, wh, reg, conf_thrs):
    B, C, H, W = heat.shape
    out = pl.pallas_call(
        _decode_kernel,
        grid=(B,),
        in_specs=[
            pl.BlockSpec((1, C, H, W), lambda b: (b, 0, 0, 0)),
            pl.BlockSpec((1, 2, H, W), lambda b: (b, 0, 0, 0)),
            pl.BlockSpec((1, 2, H, W), lambda b: (b, 0, 0, 0)),
            pl.BlockSpec(memory_space=pltpu.SMEM),
        ],
        out_specs=pl.BlockSpec((1, 8, W), lambda b: (b, 0, 0)),
        out_shape=jax.ShapeDtypeStruct((B, 8, W), jnp.float32),
        scratch_shapes=[
            pltpu.VMEM((C * H, W), jnp.float32),
            pltpu.VMEM((C, H), jnp.float32),
            pltpu.SMEM((_K,), jnp.float32),
            pltpu.SMEM((_K,), jnp.int32),
            pltpu.SMEM((_K,), jnp.int32),
            pltpu.SMEM((_K,), jnp.int32),
        ],
    )(heat, wh, reg, conf_thrs)
    return jnp.transpose(out[:, :6, :_K], (0, 2, 1))


# speedup vs baseline: 12.2647x; 1.3468x over previous
import functools

import jax
import jax.numpy as jnp
from jax.experimental import pallas as pl
from jax.experimental.pallas import tpu as pltpu

_K = 100
_BIG = 2 ** 30
_NB = 4
_CCH = 20


def _decode_kernel(heat_ref, wh_ref, reg_ref, conf_ref, out_ref, rmax_ref):
    C, H, W = 80, 128, 128
    R = C * H
    CR = _CCH * H

    rowc = jax.lax.broadcasted_iota(jnp.int32, (CR, W), 0)
    top_mask = (rowc % H == 0)
    bot_mask = (rowc % H == H - 1)
    zcol = jnp.zeros((CR, 1), jnp.float32)
    zrow = jnp.zeros((1, W), jnp.float32)

    for b in range(_NB):
        for ch in range(C // _CCH):
            sig = jax.nn.sigmoid(heat_ref[b, pl.ds(ch * CR, CR), :])
            wl = jnp.concatenate([sig[:, 1:], zcol], axis=1)
            wr = jnp.concatenate([zcol, sig[:, :-1]], axis=1)
            mw = jnp.maximum(sig, jnp.maximum(wl, wr))
            hd = jnp.concatenate([mw[1:, :], zrow], axis=0)
            hd = jnp.where(bot_mask, 0.0, hd)
            hu = jnp.concatenate([zrow, mw[:-1, :]], axis=0)
            hu = jnp.where(top_mask, 0.0, hu)
            hmax = jnp.maximum(mw, jnp.maximum(hd, hu))
            sup = jnp.where(hmax == sig, sig, 0.0)
            heat_ref[b, pl.ds(ch * CR, CR), :] = sup
            rmax_ref[pl.ds(b * C + ch * _CCH, _CCH), :] = jnp.max(
                sup.reshape(_CCH, H, W), axis=2)

    riota = jax.lax.broadcasted_iota(jnp.int32, (C, H), 0) * H + \
        jax.lax.broadcasted_iota(jnp.int32, (C, H), 1)
    lane1 = jax.lax.broadcasted_iota(jnp.int32, (1, W), 1)

    def select_body(k, carry):
        new = []
        for b in range(_NB):
            clsf_v, sc_v, x1_v, y1_v, x2_v, y2_v = carry[b]
            rm = rmax_ref[pl.ds(b * C, C), :]
            m = jnp.max(rm)
            r = jnp.min(jnp.where(rm == m, riota, _BIG))
            vals = heat_ref[b, pl.ds(r, 1), :]
            x = jnp.min(jnp.where(vals == m, lane1, _BIG))
            newrow = jnp.where(lane1 == x, 0.0, vals)
            heat_ref[b, pl.ds(r, 1), :] = newrow
            rmax_ref[pl.ds(b * C, C), :] = jnp.where(
                riota == r, jnp.max(newrow), rm)
            y = r % H
            c = r // H
            sel = (lane1 == x)
            w0 = jnp.sum(jnp.where(sel, wh_ref[b, 0, pl.ds(y, 1), :], 0.0))
            w1 = jnp.sum(jnp.where(sel, wh_ref[b, 1, pl.ds(y, 1), :], 0.0))
            r0 = jnp.sum(jnp.where(sel, reg_ref[b, 0, pl.ds(y, 1), :], 0.0))
            r1 = jnp.sum(jnp.where(sel, reg_ref[b, 1, pl.ds(y, 1), :], 0.0))
            xf = x.astype(jnp.float32) + r0
            yf = y.astype(jnp.float32) + r1
            hw = w0 * 0.5
            hh = w1 * 0.5
            clsf = jnp.where(m < conf_ref[c], -1.0, c.astype(jnp.float32))
            put = lambda acc, v: jnp.where(lane1 == k, v, acc)
            new.append((put(clsf_v, clsf), put(sc_v, m),
                        put(x1_v, xf - hw), put(y1_v, yf - hh),
                        put(x2_v, xf + hw), put(y2_v, yf + hh)))
        return tuple(new)

    z = jnp.zeros((1, W), jnp.float32)
    init = tuple((z, z, z, z, z, z) for _ in range(_NB))
    outs = jax.lax.fori_loop(0, _K, select_body, init)
    for b in range(_NB):
        out_ref[b] = jnp.concatenate(list(outs[b]) + [z, z], axis=0)


@jax.jit
def kernel(heat, wh, reg, conf_thrs):
    B, C, H, W = heat.shape
    out = pl.pallas_call(
        _decode_kernel,
        grid=(B // _NB,),
        in_specs=[
            pl.BlockSpec((_NB, C * H, W), lambda b: (b, 0, 0)),
            pl.BlockSpec((_NB, 2, H, W), lambda b: (b, 0, 0, 0)),
            pl.BlockSpec((_NB, 2, H, W), lambda b: (b, 0, 0, 0)),
            pl.BlockSpec(memory_space=pltpu.SMEM),
        ],
        out_specs=pl.BlockSpec((_NB, 8, W), lambda b: (b, 0, 0)),
        out_shape=jax.ShapeDtypeStruct((B, 8, W), jnp.float32),
        scratch_shapes=[
            pltpu.VMEM((_NB * C, H), jnp.float32),
        ],
    )(heat.reshape(B, C * H, W), wh, reg, conf_thrs)
    return jnp.transpose(out[:, :6, :_K], (0, 2, 1))


# speedup vs baseline: 13.1071x; 1.0687x over previous
import functools

import jax
import jax.numpy as jnp
from jax.experimental import pallas as pl
from jax.experimental.pallas import tpu as pltpu

_K = 100
_BIG = 2 ** 30
_NB = 4
_CCH = 20


def _decode_kernel(heat_ref, wh_ref, reg_ref, conf_ref, out_ref, rmax_ref):
    C, H, W = 80, 128, 128
    R = C * H
    CR = _CCH * H

    rowc = jax.lax.broadcasted_iota(jnp.int32, (CR, W), 0)
    top_mask = (rowc % H == 0)
    bot_mask = (rowc % H == H - 1)
    zcol = jnp.zeros((CR, 1), jnp.float32)
    zrow = jnp.zeros((1, W), jnp.float32)

    for b in range(_NB):
        for ch in range(C // _CCH):
            sig = jax.nn.sigmoid(heat_ref[b, pl.ds(ch * CR, CR), :])
            wl = jnp.concatenate([sig[:, 1:], zcol], axis=1)
            wr = jnp.concatenate([zcol, sig[:, :-1]], axis=1)
            mw = jnp.maximum(sig, jnp.maximum(wl, wr))
            hd = jnp.concatenate([mw[1:, :], zrow], axis=0)
            hd = jnp.where(bot_mask, 0.0, hd)
            hu = jnp.concatenate([zrow, mw[:-1, :]], axis=0)
            hu = jnp.where(top_mask, 0.0, hu)
            hmax = jnp.maximum(mw, jnp.maximum(hd, hu))
            sup = jnp.where(hmax == sig, sig, 0.0)
            heat_ref[b, pl.ds(ch * CR, CR), :] = sup
            rmax_ref[pl.ds(b * C + ch * _CCH, _CCH), :] = jnp.max(
                sup.reshape(_CCH, H, W), axis=2)

    riota = jax.lax.broadcasted_iota(jnp.int32, (C, H), 0) * H + \
        jax.lax.broadcasted_iota(jnp.int32, (C, H), 1)
    lane1 = jax.lax.broadcasted_iota(jnp.int32, (1, W), 1)

    def select_body(k, carry):
        new = []
        for b in range(_NB):
            rm, clsf_v, sc_v, x1_v, y1_v, x2_v, y2_v = carry[b]
            m = jnp.max(rm)
            r = jnp.min(jnp.where(rm == m, riota, _BIG))
            vals = heat_ref[b, pl.ds(r, 1), :]
            xv = jnp.min(jnp.where(vals == m, lane1, _BIG),
                         axis=1, keepdims=True)
            sel = (lane1 == xv)
            newrow = jnp.where(sel, 0.0, vals)
            heat_ref[b, pl.ds(r, 1), :] = newrow
            rm = jnp.where(riota == r, jnp.max(newrow), rm)
            x = jnp.min(xv)
            y = r % H
            c = r // H
            w0 = jnp.sum(jnp.where(sel, wh_ref[b, 0, pl.ds(y, 1), :], 0.0))
            w1 = jnp.sum(jnp.where(sel, wh_ref[b, 1, pl.ds(y, 1), :], 0.0))
            r0 = jnp.sum(jnp.where(sel, reg_ref[b, 0, pl.ds(y, 1), :], 0.0))
            r1 = jnp.sum(jnp.where(sel, reg_ref[b, 1, pl.ds(y, 1), :], 0.0))
            xf = x.astype(jnp.float32) + r0
            yf = y.astype(jnp.float32) + r1
            hw = w0 * 0.5
            hh = w1 * 0.5
            clsf = jnp.where(m < conf_ref[c], -1.0, c.astype(jnp.float32))
            put = lambda acc, v: jnp.where(lane1 == k, v, acc)
            new.append((rm, put(clsf_v, clsf), put(sc_v, m),
                        put(x1_v, xf - hw), put(y1_v, yf - hh),
                        put(x2_v, xf + hw), put(y2_v, yf + hh)))
        return tuple(new)

    z = jnp.zeros((1, W), jnp.float32)
    init = tuple((rmax_ref[pl.ds(b * C, C), :], z, z, z, z, z, z)
                 for b in range(_NB))
    outs = jax.lax.fori_loop(0, _K, select_body, init)
    for b in range(_NB):
        out_ref[b] = jnp.concatenate(list(outs[b][1:]) + [z, z], axis=0)


@jax.jit
def kernel(heat, wh, reg, conf_thrs):
    B, C, H, W = heat.shape
    out = pl.pallas_call(
        _decode_kernel,
        grid=(B // _NB,),
        in_specs=[
            pl.BlockSpec((_NB, C * H, W), lambda b: (b, 0, 0)),
            pl.BlockSpec((_NB, 2, H, W), lambda b: (b, 0, 0, 0)),
            pl.BlockSpec((_NB, 2, H, W), lambda b: (b, 0, 0, 0)),
            pl.BlockSpec(memory_space=pltpu.SMEM),
        ],
        out_specs=pl.BlockSpec((_NB, 8, W), lambda b: (b, 0, 0)),
        out_shape=jax.ShapeDtypeStruct((B, 8, W), jnp.float32),
        scratch_shapes=[
            pltpu.VMEM((_NB * C, H), jnp.float32),
        ],
    )(heat.reshape(B, C * H, W), wh, reg, conf_thrs)
    return jnp.transpose(out[:, :6, :_K], (0, 2, 1))
